# baseline (speedup 1.0000x reference)
import functools

import jax
import jax.numpy as jnp
import numpy as np
from jax import lax
from jax.experimental import pallas as pl
from jax.experimental.pallas import tpu as pltpu
from jax.experimental.pallas import tpu_sc as plsc

_N = 10000
_E = 320000
_H = 8
_F = 16
_D = _H * _F

_NC = 2
_NS = 16
_NW = _NC * _NS
_EPW = _E // _NW
_CH = 80
_CHUNKS = _EPW // _CH
_NPAD = 10240
_RPT = _NPAD // _NS

_BLK = 1000
_GRID = _N // _BLK

_R8 = np.zeros((_H, _D), np.float32)
for _hh in range(_H):
    _R8[_hh, _hh * _F:(_hh + 1) * _F] = 1.0

def _lrelu(v):
    return jnp.where(v >= 0.0, v, 0.2 * v)


def _ln(v, g, b):
    mu = jnp.mean(v, axis=-1, keepdims=True)
    var = jnp.mean((v - mu) ** 2, axis=-1, keepdims=True)
    return (v - mu) * lax.rsqrt(var + 1e-5) * g + b


def _prep_body(x_ref, w_ref, am_ref, h_ref, att_ref):
    h = jnp.dot(x_ref[...], w_ref[...], preferred_element_type=jnp.float32)
    h_ref[...] = h
    att_ref[...] = jnp.dot(h, am_ref[...], preferred_element_type=jnp.float32)


_prep_call = pl.pallas_call(
    _prep_body,
    grid=(_GRID,),
    in_specs=[
        pl.BlockSpec((_BLK, _D), lambda i: (i, 0)),
        pl.BlockSpec((_D, _D), lambda i: (0, 0)),
        pl.BlockSpec((_D, 16), lambda i: (0, 0)),
    ],
    out_specs=[
        pl.BlockSpec((_BLK, _D), lambda i: (i, 0)),
        pl.BlockSpec((_BLK, 16), lambda i: (i, 0)),
    ],
    out_shape=[
        jax.ShapeDtypeStruct((_N, _D), jnp.float32),
        jax.ShapeDtypeStruct((_N, 16), jnp.float32),
    ],
)


def _edge_body(ei_hbm, h_hbm, att_hbm,
               num_out, den_out,
               num_sh, den_sh, att_sh,
               idx0, idxd0, hbuf0, ebs0, ebd0, wbuf0,
               idx1, idxd1, hbuf1, ebs1, ebd1, wbuf1,
               sem_h0, sem_a0, sem_d0, sem_h1, sem_a1, sem_d1,
               sem_s0, sem_s1, sem_i0, sem_i1):
    cid = lax.axis_index("c")
    sid = lax.axis_index("s")
    base = sid * _RPT

    @pl.when(sid == 0)
    def _stage_att():
        pltpu.sync_copy(att_hbm, att_sh)

    zero16 = jnp.zeros((16,), jnp.float32)

    def _zn(i, c):
        hbuf0[i // 8, pl.ds((i % 8) * 16, 16)] = zero16
        return c

    lax.fori_loop(0, _CH * (_D // 16), _zn, None)

    def _zd(i, c):
        wbuf0[i, :] = zero16
        return c

    lax.fori_loop(0, _CH, _zd, None)

    for k in range(_RPT // _CH):
        pltpu.sync_copy(hbuf0, num_sh.at[pl.ds(base + k * _CH, _CH)])
        pltpu.sync_copy(wbuf0, den_sh.at[pl.ds(base + k * _CH, _CH)])
    plsc.subcore_barrier()

    ebase = (cid * _NS + sid) * _EPW
    hi = (lax.iota(jnp.int32, 16) % 8) + 8

    sets = (
        (idx0, idxd0, hbuf0, ebs0, ebd0, wbuf0,
         sem_h0, sem_a0, sem_d0, sem_s0, sem_i0),
        (idx1, idxd1, hbuf1, ebs1, ebd1, wbuf1,
         sem_h1, sem_a1, sem_d1, sem_s1, sem_i1),
    )

    def _prefetch_idx(s, ci):
        idx = s[0]
        sem_i = s[10]
        off = ebase + ci * _CH
        pltpu.async_copy(ei_hbm.at[:, pl.ds(off, _CH)], idx, sem_i)

    def _issue(s, ci):
        idx, idxd, hbuf, ebs, ebd, wbuf = s[:6]
        sem_h, sem_a, sem_d, sem_s, sem_i = s[6:]
        off = ebase + ci * _CH
        pltpu.make_async_copy(
            ei_hbm.at[:, pl.ds(off, _CH)], idx, sem_i).wait()
        pltpu.async_copy(h_hbm.at[idx.at[0]], hbuf, sem_h)
        pltpu.async_copy(att_sh.at[idx.at[0]], ebs, sem_a)
        pltpu.async_copy(att_sh.at[idx.at[1]], ebd, sem_d)

    def _process(s, ci_pref):
        idx, idxd, hbuf, ebs, ebd, wbuf = s[:6]
        sem_h, sem_a, sem_d, sem_s, sem_i = s[6:]
        pltpu.make_async_copy(h_hbm.at[idx.at[0]], hbuf, sem_h).wait()
        pltpu.make_async_copy(att_sh.at[idx.at[0]], ebs, sem_a).wait()
        pltpu.make_async_copy(att_sh.at[idx.at[1]], ebd, sem_d).wait()
        for j in range(_CH // 16):
            idxd[pl.ds(j * 16, 16)] = idx[1, pl.ds(j * 16, 16)]
        _prefetch_idx(s, ci_pref)

        @plsc.parallel_loop(0, _CH, step=1, unroll=4)
        def _wm(r):
            e = ebs[r, :] + jnp.take_along_axis(ebd[r, :], hi, axis=0)
            w = jnp.exp(jnp.where(e >= 0.0, e, 0.2 * e))
            wbuf[r, :] = w
            for hh in range(_H):
                sp = jnp.take_along_axis(
                    w, jnp.full((16,), hh, jnp.int32), axis=0)
                hbuf[r, pl.ds(hh * 16, 16)] = hbuf[r, pl.ds(hh * 16, 16)] * sp

        pltpu.async_copy(hbuf, num_sh.at[idxd], sem_s, add=True)
        pltpu.async_copy(wbuf, den_sh.at[idxd], sem_s, add=True)
        pltpu.make_async_copy(hbuf, num_sh.at[idxd], sem_s).wait()
        pltpu.make_async_copy(wbuf, den_sh.at[idxd], sem_s).wait()

    _prefetch_idx(sets[0], 0)
    _prefetch_idx(sets[1], 1)
    _issue(sets[0], 0)
    _issue(sets[1], 1)
    last = _CHUNKS - 1

    def _pair(k, c):
        ci = 2 * k
        _process(sets[0], jnp.minimum(ci + 2, last))
        _issue(sets[0], ci + 2)
        _process(sets[1], jnp.minimum(ci + 3, last))

        @pl.when(k < _CHUNKS // 2 - 1)
        def _():
            _issue(sets[1], ci + 3)

        return c

    lax.fori_loop(0, _CHUNKS // 2, _pair, None)
    _process(sets[0], last)
    pltpu.make_async_copy(
        ei_hbm.at[:, pl.ds(ebase + last * _CH, _CH)], idx0, sem_i0).wait()
    pltpu.make_async_copy(
        ei_hbm.at[:, pl.ds(ebase + last * _CH, _CH)], idx1, sem_i1).wait()
    plsc.subcore_barrier()

    pltpu.sync_copy(num_sh.at[pl.ds(base, _RPT)],
                    num_out.at[cid, pl.ds(base, _RPT)])
    pltpu.sync_copy(den_sh.at[pl.ds(base, _RPT)],
                    den_out.at[cid, pl.ds(base, _RPT)])


_edge_call = functools.partial(
    pl.kernel,
    out_type=(
        jax.ShapeDtypeStruct((_NC, _NPAD, _D), jnp.float32),
        jax.ShapeDtypeStruct((_NC, _NPAD, 16), jnp.float32),
    ),
    mesh=plsc.VectorSubcoreMesh(core_axis_name="c", subcore_axis_name="s"),
    compiler_params=pltpu.CompilerParams(use_tc_tiling_on_sc=False),
    scratch_types=[
        pltpu.MemorySpace.VMEM_SHARED((_NPAD, _D), jnp.float32),
        pltpu.MemorySpace.VMEM_SHARED((_NPAD, 16), jnp.float32),
        pltpu.MemorySpace.VMEM_SHARED((_N, 16), jnp.float32),
        pltpu.MemorySpace.VMEM((2, _CH), jnp.int32),
        pltpu.MemorySpace.VMEM((_CH,), jnp.int32),
        pltpu.MemorySpace.VMEM((_CH, _D), jnp.float32),
        pltpu.MemorySpace.VMEM((_CH, 16), jnp.float32),
        pltpu.MemorySpace.VMEM((_CH, 16), jnp.float32),
        pltpu.MemorySpace.VMEM((_CH, 16), jnp.float32),
        pltpu.MemorySpace.VMEM((2, _CH), jnp.int32),
        pltpu.MemorySpace.VMEM((_CH,), jnp.int32),
        pltpu.MemorySpace.VMEM((_CH, _D), jnp.float32),
        pltpu.MemorySpace.VMEM((_CH, 16), jnp.float32),
        pltpu.MemorySpace.VMEM((_CH, 16), jnp.float32),
        pltpu.MemorySpace.VMEM((_CH, 16), jnp.float32),
        pltpu.SemaphoreType.DMA,
        pltpu.SemaphoreType.DMA,
        pltpu.SemaphoreType.DMA,
        pltpu.SemaphoreType.DMA,
        pltpu.SemaphoreType.DMA,
        pltpu.SemaphoreType.DMA,
        pltpu.SemaphoreType.DMA,
        pltpu.SemaphoreType.DMA,
        pltpu.SemaphoreType.DMA,
        pltpu.SemaphoreType.DMA,
    ],
)(_edge_body)


def _post_body(x_ref, h_ref, att_ref, num_ref, den_ref, r8_ref,
               bg_ref, g1_ref, be1_ref, w1_ref, b1_ref, w2_ref, b2_ref,
               g2_ref, be2_ref, out_ref):
    att = att_ref[...]
    ws = jnp.exp(_lrelu(att[:, :8] + att[:, 8:]))
    den = den_ref[0][:, :8] + den_ref[1][:, :8] + ws
    r8 = r8_ref[...]
    dend = jnp.dot(den, r8, preferred_element_type=jnp.float32) + 1e-16
    wse = jnp.dot(ws, r8, preferred_element_type=jnp.float32)
    h = h_ref[...]
    numd = num_ref[0] + num_ref[1] + wse * h
    gat = numd / dend + bg_ref[...]
    u = _ln(x_ref[...] + gat, g1_ref[...], be1_ref[...])
    ff = jnp.maximum(
        jnp.dot(u, w1_ref[...], preferred_element_type=jnp.float32)
        + b1_ref[...], 0.0)
    ff = jnp.dot(ff, w2_ref[...], preferred_element_type=jnp.float32) \
        + b2_ref[...]
    out_ref[...] = _ln(u + ff, g2_ref[...], be2_ref[...])


_post_call = pl.pallas_call(
    _post_body,
    grid=(_GRID,),
    in_specs=[
        pl.BlockSpec((_BLK, _D), lambda i: (i, 0)),
        pl.BlockSpec((_BLK, _D), lambda i: (i, 0)),
        pl.BlockSpec((_BLK, 16), lambda i: (i, 0)),
        pl.BlockSpec((_NC, _BLK, _D), lambda i: (0, i, 0)),
        pl.BlockSpec((_NC, _BLK, 16), lambda i: (0, i, 0)),
        pl.BlockSpec((_H, _D), lambda i: (0, 0)),
        pl.BlockSpec((1, _D), lambda i: (0, 0)),
        pl.BlockSpec((1, _D), lambda i: (0, 0)),
        pl.BlockSpec((1, _D), lambda i: (0, 0)),
        pl.BlockSpec((_D, 2 * _D), lambda i: (0, 0)),
        pl.BlockSpec((1, 2 * _D), lambda i: (0, 0)),
        pl.BlockSpec((2 * _D, _D), lambda i: (0, 0)),
        pl.BlockSpec((1, _D), lambda i: (0, 0)),
        pl.BlockSpec((1, _D), lambda i: (0, 0)),
        pl.BlockSpec((1, _D), lambda i: (0, 0)),
    ],
    out_specs=pl.BlockSpec((_BLK, _D), lambda i: (i, 0)),
    out_shape=jax.ShapeDtypeStruct((_N, _D), jnp.float32),
)


def kernel(x, edge_index, W, a_src, a_dst, b_gat, g1, beta1, W1, b1,
           W2, b2, g2, beta2):
    eye8 = jnp.eye(_H, dtype=jnp.float32)
    a_s_m = (a_src[:, :, None] * eye8[:, None, :]).reshape(_D, _H)
    a_d_m = (a_dst[:, :, None] * eye8[:, None, :]).reshape(_D, _H)
    am = jnp.concatenate([a_s_m, a_d_m], axis=1)

    h, att = _prep_call(x, W, am)
    num2, den2 = _edge_call(edge_index, h, att)

    return _post_call(
        x, h, att, num2, den2, jnp.asarray(_R8),
        b_gat.reshape(1, _D), g1.reshape(1, _D), beta1.reshape(1, _D),
        W1, b1.reshape(1, 2 * _D), W2, b2.reshape(1, _D),
        g2.reshape(1, _D), beta2.reshape(1, _D))


# speedup vs baseline: 1.0451x; 1.0451x over previous
import functools

import jax
import jax.numpy as jnp
import numpy as np
from jax import lax
from jax.experimental import pallas as pl
from jax.experimental.pallas import tpu as pltpu
from jax.experimental.pallas import tpu_sc as plsc

_N = 10000
_E = 320000
_H = 8
_F = 16
_D = _H * _F

_NC = 2
_NS = 16
_NW = _NC * _NS
_EPW = _E // _NW
_CH = 80
_CHUNKS = _EPW // _CH
_NPAD = 10240
_RPT = _NPAD // _NS

_BLK = 1000
_GRID = _N // _BLK

_R8 = np.zeros((_H, _D), np.float32)
for _hh in range(_H):
    _R8[_hh, _hh * _F:(_hh + 1) * _F] = 1.0

def _lrelu(v):
    return jnp.where(v >= 0.0, v, 0.2 * v)


def _ln(v, g, b):
    mu = jnp.mean(v, axis=-1, keepdims=True)
    var = jnp.mean((v - mu) ** 2, axis=-1, keepdims=True)
    return (v - mu) * lax.rsqrt(var + 1e-5) * g + b


def _prep_body(x_ref, w_ref, am_ref, h_ref, att_ref):
    h = jnp.dot(x_ref[...], w_ref[...], preferred_element_type=jnp.float32)
    h_ref[...] = h
    att_ref[...] = jnp.dot(h, am_ref[...], preferred_element_type=jnp.float32)


_prep_call = pl.pallas_call(
    _prep_body,
    grid=(_GRID,),
    in_specs=[
        pl.BlockSpec((_BLK, _D), lambda i: (i, 0)),
        pl.BlockSpec((_D, _D), lambda i: (0, 0)),
        pl.BlockSpec((_D, 16), lambda i: (0, 0)),
    ],
    out_specs=[
        pl.BlockSpec((_BLK, _D), lambda i: (i, 0)),
        pl.BlockSpec((_BLK, 16), lambda i: (i, 0)),
    ],
    out_shape=[
        jax.ShapeDtypeStruct((_N, _D), jnp.float32),
        jax.ShapeDtypeStruct((_N, 16), jnp.float32),
    ],
)


def _edge_body(ei_hbm, h_hbm, att_hbm,
               num_out, den_out,
               num_sh, den_sh, att_sh,
               idx0, idxd0, hbuf0, ebs0, ebd0, wbuf0,
               idx1, idxd1, hbuf1, ebs1, ebd1, wbuf1,
               sem_h0, sem_a0, sem_d0, sem_h1, sem_a1, sem_d1,
               sem_s0, sem_s1, sem_i0, sem_i1):
    cid = lax.axis_index("c")
    sid = lax.axis_index("s")
    base = sid * _RPT

    _srows = _N // _NS
    pltpu.sync_copy(att_hbm.at[pl.ds(sid * _srows, _srows)],
                    att_sh.at[pl.ds(sid * _srows, _srows)])

    zero16 = jnp.zeros((16,), jnp.float32)

    def _zn(i, c):
        hbuf0[i // 8, pl.ds((i % 8) * 16, 16)] = zero16
        return c

    lax.fori_loop(0, _CH * (_D // 16), _zn, None)

    def _zd(i, c):
        wbuf0[i, :] = zero16
        return c

    lax.fori_loop(0, _CH, _zd, None)

    for k in range(_RPT // _CH):
        pltpu.sync_copy(hbuf0, num_sh.at[pl.ds(base + k * _CH, _CH)])
        pltpu.sync_copy(wbuf0, den_sh.at[pl.ds(base + k * _CH, _CH)])
    plsc.subcore_barrier()

    ebase = (cid * _NS + sid) * _EPW
    hi = (lax.iota(jnp.int32, 16) % 8) + 8

    sets = (
        (idx0, idxd0, hbuf0, ebs0, ebd0, wbuf0,
         sem_h0, sem_a0, sem_d0, sem_s0, sem_i0),
        (idx1, idxd1, hbuf1, ebs1, ebd1, wbuf1,
         sem_h1, sem_a1, sem_d1, sem_s1, sem_i1),
    )

    def _prefetch_idx(s, ci):
        idx = s[0]
        sem_i = s[10]
        off = ebase + ci * _CH
        pltpu.async_copy(ei_hbm.at[:, pl.ds(off, _CH)], idx, sem_i)

    def _issue(s, ci):
        idx, idxd, hbuf, ebs, ebd, wbuf = s[:6]
        sem_h, sem_a, sem_d, sem_s, sem_i = s[6:]
        off = ebase + ci * _CH
        pltpu.make_async_copy(
            ei_hbm.at[:, pl.ds(off, _CH)], idx, sem_i).wait()
        pltpu.async_copy(h_hbm.at[idx.at[0]], hbuf, sem_h)
        pltpu.async_copy(att_sh.at[idx.at[0]], ebs, sem_a)
        pltpu.async_copy(att_sh.at[idx.at[1]], ebd, sem_d)

    def _process(s, ci_pref):
        idx, idxd, hbuf, ebs, ebd, wbuf = s[:6]
        sem_h, sem_a, sem_d, sem_s, sem_i = s[6:]
        pltpu.make_async_copy(h_hbm.at[idx.at[0]], hbuf, sem_h).wait()
        pltpu.make_async_copy(att_sh.at[idx.at[0]], ebs, sem_a).wait()
        pltpu.make_async_copy(att_sh.at[idx.at[1]], ebd, sem_d).wait()
        for j in range(_CH // 16):
            idxd[pl.ds(j * 16, 16)] = idx[1, pl.ds(j * 16, 16)]
        _prefetch_idx(s, ci_pref)

        def _wm(r):
            e = ebs[r, :] + jnp.take_along_axis(ebd[r, :], hi, axis=0)
            w = jnp.exp(jnp.where(e >= 0.0, e, 0.2 * e))
            wbuf[r, :] = w
            for hh in range(_H):
                sp = jnp.take_along_axis(
                    w, jnp.full((16,), hh, jnp.int32), axis=0)
                hbuf[r, pl.ds(hh * 16, 16)] = hbuf[r, pl.ds(hh * 16, 16)] * sp

        hm = _CH // 2
        plsc.parallel_loop(0, hm, step=1, unroll=4)(_wm)
        pltpu.async_copy(hbuf.at[pl.ds(0, hm)],
                         num_sh.at[idxd.at[pl.ds(0, hm)]], sem_s, add=True)
        plsc.parallel_loop(hm, _CH, step=1, unroll=4)(_wm)
        pltpu.async_copy(hbuf.at[pl.ds(hm, hm)],
                         num_sh.at[idxd.at[pl.ds(hm, hm)]], sem_s, add=True)
        pltpu.async_copy(wbuf, den_sh.at[idxd], sem_s, add=True)
        pltpu.make_async_copy(hbuf.at[pl.ds(0, hm)],
                              num_sh.at[idxd.at[pl.ds(0, hm)]], sem_s).wait()
        pltpu.make_async_copy(hbuf.at[pl.ds(hm, hm)],
                              num_sh.at[idxd.at[pl.ds(hm, hm)]], sem_s).wait()
        pltpu.make_async_copy(wbuf, den_sh.at[idxd], sem_s).wait()

    _prefetch_idx(sets[0], 0)
    _prefetch_idx(sets[1], 1)
    _issue(sets[0], 0)
    _issue(sets[1], 1)
    last = _CHUNKS - 1

    def _pair(k, c):
        ci = 2 * k
        _process(sets[0], jnp.minimum(ci + 2, last))
        _issue(sets[0], ci + 2)
        _process(sets[1], jnp.minimum(ci + 3, last))

        @pl.when(k < _CHUNKS // 2 - 1)
        def _():
            _issue(sets[1], ci + 3)

        return c

    lax.fori_loop(0, _CHUNKS // 2, _pair, None)
    _process(sets[0], last)
    pltpu.make_async_copy(
        ei_hbm.at[:, pl.ds(ebase + last * _CH, _CH)], idx0, sem_i0).wait()
    pltpu.make_async_copy(
        ei_hbm.at[:, pl.ds(ebase + last * _CH, _CH)], idx1, sem_i1).wait()
    plsc.subcore_barrier()

    pltpu.sync_copy(num_sh.at[pl.ds(base, _RPT)],
                    num_out.at[cid, pl.ds(base, _RPT)])
    pltpu.sync_copy(den_sh.at[pl.ds(base, _RPT)],
                    den_out.at[cid, pl.ds(base, _RPT)])


_edge_call = functools.partial(
    pl.kernel,
    out_type=(
        jax.ShapeDtypeStruct((_NC, _NPAD, _D), jnp.float32),
        jax.ShapeDtypeStruct((_NC, _NPAD, 16), jnp.float32),
    ),
    mesh=plsc.VectorSubcoreMesh(core_axis_name="c", subcore_axis_name="s"),
    compiler_params=pltpu.CompilerParams(use_tc_tiling_on_sc=False),
    scratch_types=[
        pltpu.MemorySpace.VMEM_SHARED((_NPAD, _D), jnp.float32),
        pltpu.MemorySpace.VMEM_SHARED((_NPAD, 16), jnp.float32),
        pltpu.MemorySpace.VMEM_SHARED((_N, 16), jnp.float32),
        pltpu.MemorySpace.VMEM((2, _CH), jnp.int32),
        pltpu.MemorySpace.VMEM((_CH,), jnp.int32),
        pltpu.MemorySpace.VMEM((_CH, _D), jnp.float32),
        pltpu.MemorySpace.VMEM((_CH, 16), jnp.float32),
        pltpu.MemorySpace.VMEM((_CH, 16), jnp.float32),
        pltpu.MemorySpace.VMEM((_CH, 16), jnp.float32),
        pltpu.MemorySpace.VMEM((2, _CH), jnp.int32),
        pltpu.MemorySpace.VMEM((_CH,), jnp.int32),
        pltpu.MemorySpace.VMEM((_CH, _D), jnp.float32),
        pltpu.MemorySpace.VMEM((_CH, 16), jnp.float32),
        pltpu.MemorySpace.VMEM((_CH, 16), jnp.float32),
        pltpu.MemorySpace.VMEM((_CH, 16), jnp.float32),
        pltpu.SemaphoreType.DMA,
        pltpu.SemaphoreType.DMA,
        pltpu.SemaphoreType.DMA,
        pltpu.SemaphoreType.DMA,
        pltpu.SemaphoreType.DMA,
        pltpu.SemaphoreType.DMA,
        pltpu.SemaphoreType.DMA,
        pltpu.SemaphoreType.DMA,
        pltpu.SemaphoreType.DMA,
        pltpu.SemaphoreType.DMA,
    ],
)(_edge_body)


def _post_body(x_ref, h_ref, att_ref, num_ref, den_ref, r8_ref,
               bg_ref, g1_ref, be1_ref, w1_ref, b1_ref, w2_ref, b2_ref,
               g2_ref, be2_ref, out_ref):
    att = att_ref[...]
    ws = jnp.exp(_lrelu(att[:, :8] + att[:, 8:]))
    den = den_ref[0][:, :8] + den_ref[1][:, :8] + ws
    r8 = r8_ref[...]
    dend = jnp.dot(den, r8, preferred_element_type=jnp.float32) + 1e-16
    wse = jnp.dot(ws, r8, preferred_element_type=jnp.float32)
    h = h_ref[...]
    numd = num_ref[0] + num_ref[1] + wse * h
    gat = numd / dend + bg_ref[...]
    u = _ln(x_ref[...] + gat, g1_ref[...], be1_ref[...])
    ff = jnp.maximum(
        jnp.dot(u, w1_ref[...], preferred_element_type=jnp.float32)
        + b1_ref[...], 0.0)
    ff = jnp.dot(ff, w2_ref[...], preferred_element_type=jnp.float32) \
        + b2_ref[...]
    out_ref[...] = _ln(u + ff, g2_ref[...], be2_ref[...])


_post_call = pl.pallas_call(
    _post_body,
    grid=(_GRID,),
    in_specs=[
        pl.BlockSpec((_BLK, _D), lambda i: (i, 0)),
        pl.BlockSpec((_BLK, _D), lambda i: (i, 0)),
        pl.BlockSpec((_BLK, 16), lambda i: (i, 0)),
        pl.BlockSpec((_NC, _BLK, _D), lambda i: (0, i, 0)),
        pl.BlockSpec((_NC, _BLK, 16), lambda i: (0, i, 0)),
        pl.BlockSpec((_H, _D), lambda i: (0, 0)),
        pl.BlockSpec((1, _D), lambda i: (0, 0)),
        pl.BlockSpec((1, _D), lambda i: (0, 0)),
        pl.BlockSpec((1, _D), lambda i: (0, 0)),
        pl.BlockSpec((_D, 2 * _D), lambda i: (0, 0)),
        pl.BlockSpec((1, 2 * _D), lambda i: (0, 0)),
        pl.BlockSpec((2 * _D, _D), lambda i: (0, 0)),
        pl.BlockSpec((1, _D), lambda i: (0, 0)),
        pl.BlockSpec((1, _D), lambda i: (0, 0)),
        pl.BlockSpec((1, _D), lambda i: (0, 0)),
    ],
    out_specs=pl.BlockSpec((_BLK, _D), lambda i: (i, 0)),
    out_shape=jax.ShapeDtypeStruct((_N, _D), jnp.float32),
)


def kernel(x, edge_index, W, a_src, a_dst, b_gat, g1, beta1, W1, b1,
           W2, b2, g2, beta2):
    eye8 = jnp.eye(_H, dtype=jnp.float32)
    a_s_m = (a_src[:, :, None] * eye8[:, None, :]).reshape(_D, _H)
    a_d_m = (a_dst[:, :, None] * eye8[:, None, :]).reshape(_D, _H)
    am = jnp.concatenate([a_s_m, a_d_m], axis=1)

    h, att = _prep_call(x, W, am)
    num2, den2 = _edge_call(edge_index, h, att)

    return _post_call(
        x, h, att, num2, den2, jnp.asarray(_R8),
        b_gat.reshape(1, _D), g1.reshape(1, _D), beta1.reshape(1, _D),
        W1, b1.reshape(1, 2 * _D), W2, b2.reshape(1, _D),
        g2.reshape(1, _D), beta2.reshape(1, _D))


# speedup vs baseline: 1.0501x; 1.0048x over previous
import functools

import jax
import jax.numpy as jnp
import numpy as np
from jax import lax
from jax.experimental import pallas as pl
from jax.experimental.pallas import tpu as pltpu
from jax.experimental.pallas import tpu_sc as plsc

_N = 10000
_E = 320000
_H = 8
_F = 16
_D = _H * _F

_NC = 2
_NS = 16
_NW = _NC * _NS
_EPW = _E // _NW
_CH = 80
_CHUNKS = _EPW // _CH
_NPAD = 10240
_RPT = _NPAD // _NS

_BLK = 1000
_GRID = _N // _BLK

_R8 = np.zeros((_H, _D), np.float32)
for _hh in range(_H):
    _R8[_hh, _hh * _F:(_hh + 1) * _F] = 1.0

def _lrelu(v):
    return jnp.where(v >= 0.0, v, 0.2 * v)


def _ln(v, g, b):
    mu = jnp.mean(v, axis=-1, keepdims=True)
    var = jnp.mean((v - mu) ** 2, axis=-1, keepdims=True)
    return (v - mu) * lax.rsqrt(var + 1e-5) * g + b


def _prep_body(x_ref, w_ref, am_ref, h_ref, att_ref):
    h = jnp.dot(x_ref[...], w_ref[...], preferred_element_type=jnp.float32)
    h_ref[...] = h
    att_ref[...] = jnp.dot(h, am_ref[...], preferred_element_type=jnp.float32)


_prep_call = pl.pallas_call(
    _prep_body,
    grid=(_GRID,),
    in_specs=[
        pl.BlockSpec((_BLK, _D), lambda i: (i, 0)),
        pl.BlockSpec((_D, _D), lambda i: (0, 0)),
        pl.BlockSpec((_D, 16), lambda i: (0, 0)),
    ],
    out_specs=[
        pl.BlockSpec((_BLK, _D), lambda i: (i, 0)),
        pl.BlockSpec((_BLK, 16), lambda i: (i, 0)),
    ],
    out_shape=[
        jax.ShapeDtypeStruct((_N, _D), jnp.float32),
        jax.ShapeDtypeStruct((_N, 16), jnp.float32),
    ],
)


def _edge_body(ei_hbm, h_hbm, att_hbm,
               num_out, den_out,
               num_sh, den_sh, att_sh,
               idx0, idxd0, hbuf0, ebs0, ebd0, wbuf0,
               idx1, idxd1, hbuf1, ebs1, ebd1, wbuf1,
               sem_h0, sem_a0, sem_d0, sem_h1, sem_a1, sem_d1,
               sem_s0, sem_s1, sem_i0, sem_i1):
    cid = lax.axis_index("c")
    sid = lax.axis_index("s")
    base = sid * _RPT

    _srows = _N // _NS
    pltpu.sync_copy(att_hbm.at[pl.ds(sid * _srows, _srows)],
                    att_sh.at[pl.ds(sid * _srows, _srows)])

    zero16 = jnp.zeros((16,), jnp.float32)

    def _zn(i, c):
        hbuf0[i // 8, pl.ds((i % 8) * 16, 16)] = zero16
        return c

    lax.fori_loop(0, _CH * (_D // 16), _zn, None)

    def _zd(i, c):
        wbuf0[i, :] = zero16
        return c

    lax.fori_loop(0, _CH, _zd, None)

    for k in range(_RPT // _CH):
        pltpu.async_copy(hbuf0, num_sh.at[pl.ds(base + k * _CH, _CH)],
                         sem_s0)
        pltpu.async_copy(wbuf0, den_sh.at[pl.ds(base + k * _CH, _CH)],
                         sem_s0)
    for k in range(_RPT // _CH):
        pltpu.make_async_copy(
            hbuf0, num_sh.at[pl.ds(base + k * _CH, _CH)], sem_s0).wait()
        pltpu.make_async_copy(
            wbuf0, den_sh.at[pl.ds(base + k * _CH, _CH)], sem_s0).wait()
    plsc.subcore_barrier()

    ebase = (cid * _NS + sid) * _EPW
    hi = (lax.iota(jnp.int32, 16) % 8) + 8

    sets = (
        (idx0, idxd0, hbuf0, ebs0, ebd0, wbuf0,
         sem_h0, sem_a0, sem_d0, sem_s0, sem_i0),
        (idx1, idxd1, hbuf1, ebs1, ebd1, wbuf1,
         sem_h1, sem_a1, sem_d1, sem_s1, sem_i1),
    )

    def _prefetch_idx(s, ci):
        idx = s[0]
        sem_i = s[10]
        off = ebase + ci * _CH
        pltpu.async_copy(ei_hbm.at[:, pl.ds(off, _CH)], idx, sem_i)

    def _issue(s, ci):
        idx, idxd, hbuf, ebs, ebd, wbuf = s[:6]
        sem_h, sem_a, sem_d, sem_s, sem_i = s[6:]
        off = ebase + ci * _CH
        pltpu.make_async_copy(
            ei_hbm.at[:, pl.ds(off, _CH)], idx, sem_i).wait()
        pltpu.async_copy(h_hbm.at[idx.at[0]], hbuf, sem_h)
        pltpu.async_copy(att_sh.at[idx.at[0]], ebs, sem_a)
        pltpu.async_copy(att_sh.at[idx.at[1]], ebd, sem_d)

    def _process(s, ci_pref):
        idx, idxd, hbuf, ebs, ebd, wbuf = s[:6]
        sem_h, sem_a, sem_d, sem_s, sem_i = s[6:]
        pltpu.make_async_copy(h_hbm.at[idx.at[0]], hbuf, sem_h).wait()
        pltpu.make_async_copy(att_sh.at[idx.at[0]], ebs, sem_a).wait()
        pltpu.make_async_copy(att_sh.at[idx.at[1]], ebd, sem_d).wait()
        for j in range(_CH // 16):
            idxd[pl.ds(j * 16, 16)] = idx[1, pl.ds(j * 16, 16)]
        _prefetch_idx(s, ci_pref)

        def _wm(r):
            e = ebs[r, :] + jnp.take_along_axis(ebd[r, :], hi, axis=0)
            w = jnp.exp(jnp.where(e >= 0.0, e, 0.2 * e))
            wbuf[r, :] = w
            for hh in range(_H):
                sp = jnp.take_along_axis(
                    w, jnp.full((16,), hh, jnp.int32), axis=0)
                hbuf[r, pl.ds(hh * 16, 16)] = hbuf[r, pl.ds(hh * 16, 16)] * sp

        segs = ((0, 32), (32, 64), (64, _CH))
        for lo, hh2 in segs:
            plsc.parallel_loop(lo, hh2, step=1, unroll=4)(_wm)
            pltpu.async_copy(
                hbuf.at[pl.ds(lo, hh2 - lo)],
                num_sh.at[idxd.at[pl.ds(lo, hh2 - lo)]], sem_s, add=True)
        pltpu.async_copy(wbuf, den_sh.at[idxd], sem_s, add=True)
        for lo, hh2 in segs:
            pltpu.make_async_copy(
                hbuf.at[pl.ds(lo, hh2 - lo)],
                num_sh.at[idxd.at[pl.ds(lo, hh2 - lo)]], sem_s).wait()
        pltpu.make_async_copy(wbuf, den_sh.at[idxd], sem_s).wait()

    _prefetch_idx(sets[0], 0)
    _prefetch_idx(sets[1], 1)
    _issue(sets[0], 0)
    _issue(sets[1], 1)
    last = _CHUNKS - 1

    def _pair(k, c):
        ci = 2 * k
        _process(sets[0], jnp.minimum(ci + 2, last))
        _issue(sets[0], ci + 2)
        _process(sets[1], jnp.minimum(ci + 3, last))

        @pl.when(k < _CHUNKS // 2 - 1)
        def _():
            _issue(sets[1], ci + 3)

        return c

    lax.fori_loop(0, _CHUNKS // 2, _pair, None)
    _process(sets[0], last)
    pltpu.make_async_copy(
        ei_hbm.at[:, pl.ds(ebase + last * _CH, _CH)], idx0, sem_i0).wait()
    pltpu.make_async_copy(
        ei_hbm.at[:, pl.ds(ebase + last * _CH, _CH)], idx1, sem_i1).wait()
    plsc.subcore_barrier()

    pltpu.sync_copy(num_sh.at[pl.ds(base, _RPT)],
                    num_out.at[cid, pl.ds(base, _RPT)])
    pltpu.sync_copy(den_sh.at[pl.ds(base, _RPT)],
                    den_out.at[cid, pl.ds(base, _RPT)])


_edge_call = functools.partial(
    pl.kernel,
    out_type=(
        jax.ShapeDtypeStruct((_NC, _NPAD, _D), jnp.float32),
        jax.ShapeDtypeStruct((_NC, _NPAD, 16), jnp.float32),
    ),
    mesh=plsc.VectorSubcoreMesh(core_axis_name="c", subcore_axis_name="s"),
    compiler_params=pltpu.CompilerParams(use_tc_tiling_on_sc=False),
    scratch_types=[
        pltpu.MemorySpace.VMEM_SHARED((_NPAD, _D), jnp.float32),
        pltpu.MemorySpace.VMEM_SHARED((_NPAD, 16), jnp.float32),
        pltpu.MemorySpace.VMEM_SHARED((_N, 16), jnp.float32),
        pltpu.MemorySpace.VMEM((2, _CH), jnp.int32),
        pltpu.MemorySpace.VMEM((_CH,), jnp.int32),
        pltpu.MemorySpace.VMEM((_CH, _D), jnp.float32),
        pltpu.MemorySpace.VMEM((_CH, 16), jnp.float32),
        pltpu.MemorySpace.VMEM((_CH, 16), jnp.float32),
        pltpu.MemorySpace.VMEM((_CH, 16), jnp.float32),
        pltpu.MemorySpace.VMEM((2, _CH), jnp.int32),
        pltpu.MemorySpace.VMEM((_CH,), jnp.int32),
        pltpu.MemorySpace.VMEM((_CH, _D), jnp.float32),
        pltpu.MemorySpace.VMEM((_CH, 16), jnp.float32),
        pltpu.MemorySpace.VMEM((_CH, 16), jnp.float32),
        pltpu.MemorySpace.VMEM((_CH, 16), jnp.float32),
        pltpu.SemaphoreType.DMA,
        pltpu.SemaphoreType.DMA,
        pltpu.SemaphoreType.DMA,
        pltpu.SemaphoreType.DMA,
        pltpu.SemaphoreType.DMA,
        pltpu.SemaphoreType.DMA,
        pltpu.SemaphoreType.DMA,
        pltpu.SemaphoreType.DMA,
        pltpu.SemaphoreType.DMA,
        pltpu.SemaphoreType.DMA,
    ],
)(_edge_body)


def _post_body(x_ref, h_ref, att_ref, num_ref, den_ref, r8_ref,
               bg_ref, g1_ref, be1_ref, w1_ref, b1_ref, w2_ref, b2_ref,
               g2_ref, be2_ref, out_ref):
    att = att_ref[...]
    ws = jnp.exp(_lrelu(att[:, :8] + att[:, 8:]))
    den = den_ref[0][:, :8] + den_ref[1][:, :8] + ws
    r8 = r8_ref[...]
    dend = jnp.dot(den, r8, preferred_element_type=jnp.float32) + 1e-16
    wse = jnp.dot(ws, r8, preferred_element_type=jnp.float32)
    h = h_ref[...]
    numd = num_ref[0] + num_ref[1] + wse * h
    gat = numd / dend + bg_ref[...]
    u = _ln(x_ref[...] + gat, g1_ref[...], be1_ref[...])
    ff = jnp.maximum(
        jnp.dot(u, w1_ref[...], preferred_element_type=jnp.float32)
        + b1_ref[...], 0.0)
    ff = jnp.dot(ff, w2_ref[...], preferred_element_type=jnp.float32) \
        + b2_ref[...]
    out_ref[...] = _ln(u + ff, g2_ref[...], be2_ref[...])


_post_call = pl.pallas_call(
    _post_body,
    grid=(_GRID,),
    in_specs=[
        pl.BlockSpec((_BLK, _D), lambda i: (i, 0)),
        pl.BlockSpec((_BLK, _D), lambda i: (i, 0)),
        pl.BlockSpec((_BLK, 16), lambda i: (i, 0)),
        pl.BlockSpec((_NC, _BLK, _D), lambda i: (0, i, 0)),
        pl.BlockSpec((_NC, _BLK, 16), lambda i: (0, i, 0)),
        pl.BlockSpec((_H, _D), lambda i: (0, 0)),
        pl.BlockSpec((1, _D), lambda i: (0, 0)),
        pl.BlockSpec((1, _D), lambda i: (0, 0)),
        pl.BlockSpec((1, _D), lambda i: (0, 0)),
        pl.BlockSpec((_D, 2 * _D), lambda i: (0, 0)),
        pl.BlockSpec((1, 2 * _D), lambda i: (0, 0)),
        pl.BlockSpec((2 * _D, _D), lambda i: (0, 0)),
        pl.BlockSpec((1, _D), lambda i: (0, 0)),
        pl.BlockSpec((1, _D), lambda i: (0, 0)),
        pl.BlockSpec((1, _D), lambda i: (0, 0)),
    ],
    out_specs=pl.BlockSpec((_BLK, _D), lambda i: (i, 0)),
    out_shape=jax.ShapeDtypeStruct((_N, _D), jnp.float32),
)


def kernel(x, edge_index, W, a_src, a_dst, b_gat, g1, beta1, W1, b1,
           W2, b2, g2, beta2):
    eye8 = jnp.eye(_H, dtype=jnp.float32)
    a_s_m = (a_src[:, :, None] * eye8[:, None, :]).reshape(_D, _H)
    a_d_m = (a_dst[:, :, None] * eye8[:, None, :]).reshape(_D, _H)
    am = jnp.concatenate([a_s_m, a_d_m], axis=1)

    h, att = _prep_call(x, W, am)
    num2, den2 = _edge_call(edge_index, h, att)

    return _post_call(
        x, h, att, num2, den2, jnp.asarray(_R8),
        b_gat.reshape(1, _D), g1.reshape(1, _D), beta1.reshape(1, _D),
        W1, b1.reshape(1, 2 * _D), W2, b2.reshape(1, _D),
        g2.reshape(1, _D), beta2.reshape(1, _D))


# speedup vs baseline: 1.0815x; 1.0299x over previous
import functools

import jax
import jax.numpy as jnp
import numpy as np
from jax import lax
from jax.experimental import pallas as pl
from jax.experimental.pallas import tpu as pltpu
from jax.experimental.pallas import tpu_sc as plsc

_N = 10000
_E = 320000
_H = 8
_F = 16
_D = _H * _F

_NC = 2
_NS = 16
_NW = _NC * _NS
_EPW = _E // _NW
_CH = 80
_CHUNKS = _EPW // _CH
_NPAD = 10240
_RPT = _NPAD // _NS

_BLK = 2000
_GRID = _N // _BLK

_R8 = np.zeros((_H, _D), np.float32)
for _hh in range(_H):
    _R8[_hh, _hh * _F:(_hh + 1) * _F] = 1.0

def _lrelu(v):
    return jnp.where(v >= 0.0, v, 0.2 * v)


def _ln(v, g, b):
    mu = jnp.mean(v, axis=-1, keepdims=True)
    var = jnp.mean((v - mu) ** 2, axis=-1, keepdims=True)
    return (v - mu) * lax.rsqrt(var + 1e-5) * g + b


def _prep_body(x_ref, w_ref, am_ref, h_ref, att_ref):
    h = jnp.dot(x_ref[...], w_ref[...], preferred_element_type=jnp.float32)
    h_ref[...] = h
    att_ref[...] = jnp.dot(h, am_ref[...], preferred_element_type=jnp.float32)


_prep_call = pl.pallas_call(
    _prep_body,
    grid=(_GRID,),
    in_specs=[
        pl.BlockSpec((_BLK, _D), lambda i: (i, 0)),
        pl.BlockSpec((_D, _D), lambda i: (0, 0)),
        pl.BlockSpec((_D, 16), lambda i: (0, 0)),
    ],
    out_specs=[
        pl.BlockSpec((_BLK, _D), lambda i: (i, 0)),
        pl.BlockSpec((_BLK, 16), lambda i: (i, 0)),
    ],
    out_shape=[
        jax.ShapeDtypeStruct((_N, _D), jnp.float32),
        jax.ShapeDtypeStruct((_N, 16), jnp.float32),
    ],
)


def _edge_body(ei_hbm, h_hbm, att_hbm,
               num_out, den_out,
               num_sh, den_sh, att_sh,
               idx0, idxd0, hbuf0, ebs0, ebd0, wbuf0,
               idx1, idxd1, hbuf1, ebs1, ebd1, wbuf1,
               sem_h0, sem_a0, sem_d0, sem_h1, sem_a1, sem_d1,
               sem_s0, sem_s1, sem_i0, sem_i1):
    cid = lax.axis_index("c")
    sid = lax.axis_index("s")
    base = sid * _RPT

    _srows = _N // _NS
    pltpu.sync_copy(att_hbm.at[pl.ds(sid * _srows, _srows)],
                    att_sh.at[pl.ds(sid * _srows, _srows)])

    zero16 = jnp.zeros((16,), jnp.float32)

    def _zn(i, c):
        hbuf0[i // 8, pl.ds((i % 8) * 16, 16)] = zero16
        return c

    lax.fori_loop(0, _CH * (_D // 16), _zn, None)

    def _zd(i, c):
        wbuf0[i, :] = zero16
        return c

    lax.fori_loop(0, _CH, _zd, None)

    for k in range(_RPT // _CH):
        pltpu.async_copy(hbuf0, num_sh.at[pl.ds(base + k * _CH, _CH)],
                         sem_s0)
        pltpu.async_copy(wbuf0, den_sh.at[pl.ds(base + k * _CH, _CH)],
                         sem_s0)
    for k in range(_RPT // _CH):
        pltpu.make_async_copy(
            hbuf0, num_sh.at[pl.ds(base + k * _CH, _CH)], sem_s0).wait()
        pltpu.make_async_copy(
            wbuf0, den_sh.at[pl.ds(base + k * _CH, _CH)], sem_s0).wait()
    plsc.subcore_barrier()

    ebase = (cid * _NS + sid) * _EPW
    hi = (lax.iota(jnp.int32, 16) % 8) + 8

    sets = (
        (idx0, idxd0, hbuf0, ebs0, ebd0, wbuf0,
         sem_h0, sem_a0, sem_d0, sem_s0, sem_i0),
        (idx1, idxd1, hbuf1, ebs1, ebd1, wbuf1,
         sem_h1, sem_a1, sem_d1, sem_s1, sem_i1),
    )

    def _prefetch_idx(s, ci):
        idx = s[0]
        sem_i = s[10]
        off = ebase + ci * _CH
        pltpu.async_copy(ei_hbm.at[:, pl.ds(off, _CH)], idx, sem_i)

    def _issue(s, ci):
        idx, idxd, hbuf, ebs, ebd, wbuf = s[:6]
        sem_h, sem_a, sem_d, sem_s, sem_i = s[6:]
        off = ebase + ci * _CH
        pltpu.make_async_copy(
            ei_hbm.at[:, pl.ds(off, _CH)], idx, sem_i).wait()
        pltpu.async_copy(h_hbm.at[idx.at[0]], hbuf, sem_h)
        pltpu.async_copy(att_sh.at[idx.at[0]], ebs, sem_a)
        pltpu.async_copy(att_sh.at[idx.at[1]], ebd, sem_d)

    def _process(s, ci_pref):
        idx, idxd, hbuf, ebs, ebd, wbuf = s[:6]
        sem_h, sem_a, sem_d, sem_s, sem_i = s[6:]
        pltpu.make_async_copy(h_hbm.at[idx.at[0]], hbuf, sem_h).wait()
        pltpu.make_async_copy(att_sh.at[idx.at[0]], ebs, sem_a).wait()
        pltpu.make_async_copy(att_sh.at[idx.at[1]], ebd, sem_d).wait()
        for j in range(_CH // 16):
            idxd[pl.ds(j * 16, 16)] = idx[1, pl.ds(j * 16, 16)]
        _prefetch_idx(s, ci_pref)

        def _wm(r):
            e = ebs[r, :] + jnp.take_along_axis(ebd[r, :], hi, axis=0)
            w = jnp.exp(jnp.where(e >= 0.0, e, 0.2 * e))
            wbuf[r, :] = w
            for hh in range(_H):
                sp = jnp.take_along_axis(
                    w, jnp.full((16,), hh, jnp.int32), axis=0)
                hbuf[r, pl.ds(hh * 16, 16)] = hbuf[r, pl.ds(hh * 16, 16)] * sp

        segs = ((0, 32), (32, 64), (64, _CH))
        for lo, hh2 in segs:
            plsc.parallel_loop(lo, hh2, step=1, unroll=4)(_wm)
            pltpu.async_copy(
                hbuf.at[pl.ds(lo, hh2 - lo)],
                num_sh.at[idxd.at[pl.ds(lo, hh2 - lo)]], sem_s, add=True)
        pltpu.async_copy(wbuf, den_sh.at[idxd], sem_s, add=True)
        for lo, hh2 in segs:
            pltpu.make_async_copy(
                hbuf.at[pl.ds(lo, hh2 - lo)],
                num_sh.at[idxd.at[pl.ds(lo, hh2 - lo)]], sem_s).wait()
        pltpu.make_async_copy(wbuf, den_sh.at[idxd], sem_s).wait()

    _prefetch_idx(sets[0], 0)
    _prefetch_idx(sets[1], 1)
    _issue(sets[0], 0)
    _issue(sets[1], 1)
    last = _CHUNKS - 1

    def _pair(k, c):
        ci = 2 * k
        _process(sets[0], jnp.minimum(ci + 2, last))
        _issue(sets[0], ci + 2)
        _process(sets[1], jnp.minimum(ci + 3, last))

        @pl.when(k < _CHUNKS // 2 - 1)
        def _():
            _issue(sets[1], ci + 3)

        return c

    lax.fori_loop(0, _CHUNKS // 2, _pair, None)
    _process(sets[0], last)
    pltpu.make_async_copy(
        ei_hbm.at[:, pl.ds(ebase + last * _CH, _CH)], idx0, sem_i0).wait()
    pltpu.make_async_copy(
        ei_hbm.at[:, pl.ds(ebase + last * _CH, _CH)], idx1, sem_i1).wait()
    plsc.subcore_barrier()

    pltpu.sync_copy(num_sh.at[pl.ds(base, _RPT)],
                    num_out.at[cid, pl.ds(base, _RPT)])
    pltpu.sync_copy(den_sh.at[pl.ds(base, _RPT)],
                    den_out.at[cid, pl.ds(base, _RPT)])


_edge_call = functools.partial(
    pl.kernel,
    out_type=(
        jax.ShapeDtypeStruct((_NC, _NPAD, _D), jnp.float32),
        jax.ShapeDtypeStruct((_NC, _NPAD, 16), jnp.float32),
    ),
    mesh=plsc.VectorSubcoreMesh(core_axis_name="c", subcore_axis_name="s"),
    compiler_params=pltpu.CompilerParams(use_tc_tiling_on_sc=False),
    scratch_types=[
        pltpu.MemorySpace.VMEM_SHARED((_NPAD, _D), jnp.float32),
        pltpu.MemorySpace.VMEM_SHARED((_NPAD, 16), jnp.float32),
        pltpu.MemorySpace.VMEM_SHARED((_N, 16), jnp.float32),
        pltpu.MemorySpace.VMEM((2, _CH), jnp.int32),
        pltpu.MemorySpace.VMEM((_CH,), jnp.int32),
        pltpu.MemorySpace.VMEM((_CH, _D), jnp.float32),
        pltpu.MemorySpace.VMEM((_CH, 16), jnp.float32),
        pltpu.MemorySpace.VMEM((_CH, 16), jnp.float32),
        pltpu.MemorySpace.VMEM((_CH, 16), jnp.float32),
        pltpu.MemorySpace.VMEM((2, _CH), jnp.int32),
        pltpu.MemorySpace.VMEM((_CH,), jnp.int32),
        pltpu.MemorySpace.VMEM((_CH, _D), jnp.float32),
        pltpu.MemorySpace.VMEM((_CH, 16), jnp.float32),
        pltpu.MemorySpace.VMEM((_CH, 16), jnp.float32),
        pltpu.MemorySpace.VMEM((_CH, 16), jnp.float32),
        pltpu.SemaphoreType.DMA,
        pltpu.SemaphoreType.DMA,
        pltpu.SemaphoreType.DMA,
        pltpu.SemaphoreType.DMA,
        pltpu.SemaphoreType.DMA,
        pltpu.SemaphoreType.DMA,
        pltpu.SemaphoreType.DMA,
        pltpu.SemaphoreType.DMA,
        pltpu.SemaphoreType.DMA,
        pltpu.SemaphoreType.DMA,
    ],
)(_edge_body)


def _post_body(x_ref, h_ref, att_ref, num_ref, den_ref, r8_ref,
               bg_ref, g1_ref, be1_ref, w1_ref, b1_ref, w2_ref, b2_ref,
               g2_ref, be2_ref, out_ref):
    att = att_ref[...]
    ws = jnp.exp(_lrelu(att[:, :8] + att[:, 8:]))
    den = den_ref[0][:, :8] + den_ref[1][:, :8] + ws
    r8 = r8_ref[...]
    dend = jnp.dot(den, r8, preferred_element_type=jnp.float32) + 1e-16
    wse = jnp.dot(ws, r8, preferred_element_type=jnp.float32)
    h = h_ref[...]
    numd = num_ref[0] + num_ref[1] + wse * h
    gat = numd / dend + bg_ref[...]
    u = _ln(x_ref[...] + gat, g1_ref[...], be1_ref[...])
    ff = jnp.maximum(
        jnp.dot(u, w1_ref[...], preferred_element_type=jnp.float32)
        + b1_ref[...], 0.0)
    ff = jnp.dot(ff, w2_ref[...], preferred_element_type=jnp.float32) \
        + b2_ref[...]
    out_ref[...] = _ln(u + ff, g2_ref[...], be2_ref[...])


_post_call = pl.pallas_call(
    _post_body,
    grid=(_GRID,),
    in_specs=[
        pl.BlockSpec((_BLK, _D), lambda i: (i, 0)),
        pl.BlockSpec((_BLK, _D), lambda i: (i, 0)),
        pl.BlockSpec((_BLK, 16), lambda i: (i, 0)),
        pl.BlockSpec((_NC, _BLK, _D), lambda i: (0, i, 0)),
        pl.BlockSpec((_NC, _BLK, 16), lambda i: (0, i, 0)),
        pl.BlockSpec((_H, _D), lambda i: (0, 0)),
        pl.BlockSpec((1, _D), lambda i: (0, 0)),
        pl.BlockSpec((1, _D), lambda i: (0, 0)),
        pl.BlockSpec((1, _D), lambda i: (0, 0)),
        pl.BlockSpec((_D, 2 * _D), lambda i: (0, 0)),
        pl.BlockSpec((1, 2 * _D), lambda i: (0, 0)),
        pl.BlockSpec((2 * _D, _D), lambda i: (0, 0)),
        pl.BlockSpec((1, _D), lambda i: (0, 0)),
        pl.BlockSpec((1, _D), lambda i: (0, 0)),
        pl.BlockSpec((1, _D), lambda i: (0, 0)),
    ],
    out_specs=pl.BlockSpec((_BLK, _D), lambda i: (i, 0)),
    out_shape=jax.ShapeDtypeStruct((_N, _D), jnp.float32),
)


def kernel(x, edge_index, W, a_src, a_dst, b_gat, g1, beta1, W1, b1,
           W2, b2, g2, beta2):
    eye8 = jnp.eye(_H, dtype=jnp.float32)
    a_s_m = (a_src[:, :, None] * eye8[:, None, :]).reshape(_D, _H)
    a_d_m = (a_dst[:, :, None] * eye8[:, None, :]).reshape(_D, _H)
    am = jnp.concatenate([a_s_m, a_d_m], axis=1)

    h, att = _prep_call(x, W, am)
    num2, den2 = _edge_call(edge_index, h, att)

    return _post_call(
        x, h, att, num2, den2, jnp.asarray(_R8),
        b_gat.reshape(1, _D), g1.reshape(1, _D), beta1.reshape(1, _D),
        W1, b1.reshape(1, 2 * _D), W2, b2.reshape(1, _D),
        g2.reshape(1, _D), beta2.reshape(1, _D))


# speedup vs baseline: 1.0896x; 1.0075x over previous
import functools

import jax
import jax.numpy as jnp
import numpy as np
from jax import lax
from jax.experimental import pallas as pl
from jax.experimental.pallas import tpu as pltpu
from jax.experimental.pallas import tpu_sc as plsc

_N = 10000
_E = 320000
_H = 8
_F = 16
_D = _H * _F

_NC = 2
_NS = 16
_NW = _NC * _NS
_EPW = _E // _NW
_CH = 80
_CHUNKS = _EPW // _CH
_NPAD = 10240
_RPT = _NPAD // _NS

_BLK = 5000
_GRID = _N // _BLK

_R8 = np.zeros((_H, _D), np.float32)
for _hh in range(_H):
    _R8[_hh, _hh * _F:(_hh + 1) * _F] = 1.0

def _lrelu(v):
    return jnp.where(v >= 0.0, v, 0.2 * v)


def _ln(v, g, b):
    mu = jnp.mean(v, axis=-1, keepdims=True)
    var = jnp.mean((v - mu) ** 2, axis=-1, keepdims=True)
    return (v - mu) * lax.rsqrt(var + 1e-5) * g + b


def _prep_body(x_ref, w_ref, am_ref, h_ref, att_ref):
    h = jnp.dot(x_ref[...], w_ref[...], preferred_element_type=jnp.float32)
    h_ref[...] = h
    att_ref[...] = jnp.dot(h, am_ref[...], preferred_element_type=jnp.float32)


_prep_call = pl.pallas_call(
    _prep_body,
    grid=(_GRID,),
    in_specs=[
        pl.BlockSpec((_BLK, _D), lambda i: (i, 0)),
        pl.BlockSpec((_D, _D), lambda i: (0, 0)),
        pl.BlockSpec((_D, 16), lambda i: (0, 0)),
    ],
    out_specs=[
        pl.BlockSpec((_BLK, _D), lambda i: (i, 0)),
        pl.BlockSpec((_BLK, 16), lambda i: (i, 0)),
    ],
    out_shape=[
        jax.ShapeDtypeStruct((_N, _D), jnp.float32),
        jax.ShapeDtypeStruct((_N, 16), jnp.float32),
    ],
)


def _edge_body(ei_hbm, h_hbm, att_hbm,
               num_out, den_out,
               num_sh, den_sh, att_sh,
               idx0, idxd0, hbuf0, ebs0, ebd0, wbuf0,
               idx1, idxd1, hbuf1, ebs1, ebd1, wbuf1,
               sem_h0, sem_a0, sem_d0, sem_h1, sem_a1, sem_d1,
               sem_s0, sem_s1, sem_i0, sem_i1):
    cid = lax.axis_index("c")
    sid = lax.axis_index("s")
    base = sid * _RPT

    _srows = _N // _NS
    pltpu.sync_copy(att_hbm.at[pl.ds(sid * _srows, _srows)],
                    att_sh.at[pl.ds(sid * _srows, _srows)])

    zero16 = jnp.zeros((16,), jnp.float32)

    def _zn(i, c):
        hbuf0[i // 8, pl.ds((i % 8) * 16, 16)] = zero16
        return c

    lax.fori_loop(0, _CH * (_D // 16), _zn, None)

    def _zd(i, c):
        wbuf0[i, :] = zero16
        return c

    lax.fori_loop(0, _CH, _zd, None)

    for k in range(_RPT // _CH):
        pltpu.async_copy(hbuf0, num_sh.at[pl.ds(base + k * _CH, _CH)],
                         sem_s0)
        pltpu.async_copy(wbuf0, den_sh.at[pl.ds(base + k * _CH, _CH)],
                         sem_s0)
    for k in range(_RPT // _CH):
        pltpu.make_async_copy(
            hbuf0, num_sh.at[pl.ds(base + k * _CH, _CH)], sem_s0).wait()
        pltpu.make_async_copy(
            wbuf0, den_sh.at[pl.ds(base + k * _CH, _CH)], sem_s0).wait()
    plsc.subcore_barrier()

    ebase = (cid * _NS + sid) * _EPW
    hi = (lax.iota(jnp.int32, 16) % 8) + 8

    sets = (
        (idx0, idxd0, hbuf0, ebs0, ebd0, wbuf0,
         sem_h0, sem_a0, sem_d0, sem_s0, sem_i0),
        (idx1, idxd1, hbuf1, ebs1, ebd1, wbuf1,
         sem_h1, sem_a1, sem_d1, sem_s1, sem_i1),
    )

    def _prefetch_idx(s, ci):
        idx = s[0]
        sem_i = s[10]
        off = ebase + ci * _CH
        pltpu.async_copy(ei_hbm.at[:, pl.ds(off, _CH)], idx, sem_i)

    def _issue(s, ci):
        idx, idxd, hbuf, ebs, ebd, wbuf = s[:6]
        sem_h, sem_a, sem_d, sem_s, sem_i = s[6:]
        off = ebase + ci * _CH
        pltpu.make_async_copy(
            ei_hbm.at[:, pl.ds(off, _CH)], idx, sem_i).wait()
        pltpu.async_copy(h_hbm.at[idx.at[0]], hbuf, sem_h)
        pltpu.async_copy(att_sh.at[idx.at[0]], ebs, sem_a)
        pltpu.async_copy(att_sh.at[idx.at[1]], ebd, sem_d)

    def _process(s, ci_pref):
        idx, idxd, hbuf, ebs, ebd, wbuf = s[:6]
        sem_h, sem_a, sem_d, sem_s, sem_i = s[6:]
        pltpu.make_async_copy(h_hbm.at[idx.at[0]], hbuf, sem_h).wait()
        pltpu.make_async_copy(att_sh.at[idx.at[0]], ebs, sem_a).wait()
        pltpu.make_async_copy(att_sh.at[idx.at[1]], ebd, sem_d).wait()
        for j in range(_CH // 16):
            idxd[pl.ds(j * 16, 16)] = idx[1, pl.ds(j * 16, 16)]
        _prefetch_idx(s, ci_pref)

        def _wm(r):
            e = ebs[r, :] + jnp.take_along_axis(ebd[r, :], hi, axis=0)
            w = jnp.exp(jnp.where(e >= 0.0, e, 0.2 * e))
            wbuf[r, :] = w
            for hh in range(_H):
                sp = jnp.take_along_axis(
                    w, jnp.full((16,), hh, jnp.int32), axis=0)
                hbuf[r, pl.ds(hh * 16, 16)] = hbuf[r, pl.ds(hh * 16, 16)] * sp

        segs = ((0, 32), (32, 64), (64, _CH))
        for lo, hh2 in segs:
            plsc.parallel_loop(lo, hh2, step=1, unroll=4)(_wm)
            pltpu.async_copy(
                hbuf.at[pl.ds(lo, hh2 - lo)],
                num_sh.at[idxd.at[pl.ds(lo, hh2 - lo)]], sem_s, add=True)
        pltpu.async_copy(wbuf, den_sh.at[idxd], sem_s, add=True)
        for lo, hh2 in segs:
            pltpu.make_async_copy(
                hbuf.at[pl.ds(lo, hh2 - lo)],
                num_sh.at[idxd.at[pl.ds(lo, hh2 - lo)]], sem_s).wait()
        pltpu.make_async_copy(wbuf, den_sh.at[idxd], sem_s).wait()

    _prefetch_idx(sets[0], 0)
    _prefetch_idx(sets[1], 1)
    _issue(sets[0], 0)
    _issue(sets[1], 1)
    last = _CHUNKS - 1

    def _pair(k, c):
        ci = 2 * k
        _process(sets[0], jnp.minimum(ci + 2, last))
        _issue(sets[0], ci + 2)
        _process(sets[1], jnp.minimum(ci + 3, last))

        @pl.when(k < _CHUNKS // 2 - 1)
        def _():
            _issue(sets[1], ci + 3)

        return c

    lax.fori_loop(0, _CHUNKS // 2, _pair, None)
    _process(sets[0], last)
    pltpu.make_async_copy(
        ei_hbm.at[:, pl.ds(ebase + last * _CH, _CH)], idx0, sem_i0).wait()
    pltpu.make_async_copy(
        ei_hbm.at[:, pl.ds(ebase + last * _CH, _CH)], idx1, sem_i1).wait()
    plsc.subcore_barrier()

    pltpu.sync_copy(num_sh.at[pl.ds(base, _RPT)],
                    num_out.at[cid, pl.ds(base, _RPT)])
    pltpu.sync_copy(den_sh.at[pl.ds(base, _RPT)],
                    den_out.at[cid, pl.ds(base, _RPT)])


_edge_call = functools.partial(
    pl.kernel,
    out_type=(
        jax.ShapeDtypeStruct((_NC, _NPAD, _D), jnp.float32),
        jax.ShapeDtypeStruct((_NC, _NPAD, 16), jnp.float32),
    ),
    mesh=plsc.VectorSubcoreMesh(core_axis_name="c", subcore_axis_name="s"),
    compiler_params=pltpu.CompilerParams(use_tc_tiling_on_sc=False),
    scratch_types=[
        pltpu.MemorySpace.VMEM_SHARED((_NPAD, _D), jnp.float32),
        pltpu.MemorySpace.VMEM_SHARED((_NPAD, 16), jnp.float32),
        pltpu.MemorySpace.VMEM_SHARED((_N, 16), jnp.float32),
        pltpu.MemorySpace.VMEM((2, _CH), jnp.int32),
        pltpu.MemorySpace.VMEM((_CH,), jnp.int32),
        pltpu.MemorySpace.VMEM((_CH, _D), jnp.float32),
        pltpu.MemorySpace.VMEM((_CH, 16), jnp.float32),
        pltpu.MemorySpace.VMEM((_CH, 16), jnp.float32),
        pltpu.MemorySpace.VMEM((_CH, 16), jnp.float32),
        pltpu.MemorySpace.VMEM((2, _CH), jnp.int32),
        pltpu.MemorySpace.VMEM((_CH,), jnp.int32),
        pltpu.MemorySpace.VMEM((_CH, _D), jnp.float32),
        pltpu.MemorySpace.VMEM((_CH, 16), jnp.float32),
        pltpu.MemorySpace.VMEM((_CH, 16), jnp.float32),
        pltpu.MemorySpace.VMEM((_CH, 16), jnp.float32),
        pltpu.SemaphoreType.DMA,
        pltpu.SemaphoreType.DMA,
        pltpu.SemaphoreType.DMA,
        pltpu.SemaphoreType.DMA,
        pltpu.SemaphoreType.DMA,
        pltpu.SemaphoreType.DMA,
        pltpu.SemaphoreType.DMA,
        pltpu.SemaphoreType.DMA,
        pltpu.SemaphoreType.DMA,
        pltpu.SemaphoreType.DMA,
    ],
)(_edge_body)


def _post_body(x_ref, h_ref, att_ref, num_ref, den_ref, r8_ref,
               bg_ref, g1_ref, be1_ref, w1_ref, b1_ref, w2_ref, b2_ref,
               g2_ref, be2_ref, out_ref):
    att = att_ref[...]
    ws = jnp.exp(_lrelu(att[:, :8] + att[:, 8:]))
    den = den_ref[0][:, :8] + den_ref[1][:, :8] + ws
    r8 = r8_ref[...]
    dend = jnp.dot(den, r8, preferred_element_type=jnp.float32) + 1e-16
    wse = jnp.dot(ws, r8, preferred_element_type=jnp.float32)
    h = h_ref[...]
    numd = num_ref[0] + num_ref[1] + wse * h
    gat = numd / dend + bg_ref[...]
    u = _ln(x_ref[...] + gat, g1_ref[...], be1_ref[...])
    ff = jnp.maximum(
        jnp.dot(u, w1_ref[...], preferred_element_type=jnp.float32)
        + b1_ref[...], 0.0)
    ff = jnp.dot(ff, w2_ref[...], preferred_element_type=jnp.float32) \
        + b2_ref[...]
    out_ref[...] = _ln(u + ff, g2_ref[...], be2_ref[...])


_post_call = pl.pallas_call(
    _post_body,
    grid=(_GRID,),
    in_specs=[
        pl.BlockSpec((_BLK, _D), lambda i: (i, 0)),
        pl.BlockSpec((_BLK, _D), lambda i: (i, 0)),
        pl.BlockSpec((_BLK, 16), lambda i: (i, 0)),
        pl.BlockSpec((_NC, _BLK, _D), lambda i: (0, i, 0)),
        pl.BlockSpec((_NC, _BLK, 16), lambda i: (0, i, 0)),
        pl.BlockSpec((_H, _D), lambda i: (0, 0)),
        pl.BlockSpec((1, _D), lambda i: (0, 0)),
        pl.BlockSpec((1, _D), lambda i: (0, 0)),
        pl.BlockSpec((1, _D), lambda i: (0, 0)),
        pl.BlockSpec((_D, 2 * _D), lambda i: (0, 0)),
        pl.BlockSpec((1, 2 * _D), lambda i: (0, 0)),
        pl.BlockSpec((2 * _D, _D), lambda i: (0, 0)),
        pl.BlockSpec((1, _D), lambda i: (0, 0)),
        pl.BlockSpec((1, _D), lambda i: (0, 0)),
        pl.BlockSpec((1, _D), lambda i: (0, 0)),
    ],
    out_specs=pl.BlockSpec((_BLK, _D), lambda i: (i, 0)),
    out_shape=jax.ShapeDtypeStruct((_N, _D), jnp.float32),
)


def kernel(x, edge_index, W, a_src, a_dst, b_gat, g1, beta1, W1, b1,
           W2, b2, g2, beta2):
    eye8 = jnp.eye(_H, dtype=jnp.float32)
    a_s_m = (a_src[:, :, None] * eye8[:, None, :]).reshape(_D, _H)
    a_d_m = (a_dst[:, :, None] * eye8[:, None, :]).reshape(_D, _H)
    am = jnp.concatenate([a_s_m, a_d_m], axis=1)

    h, att = _prep_call(x, W, am)
    num2, den2 = _edge_call(edge_index, h, att)

    return _post_call(
        x, h, att, num2, den2, jnp.asarray(_R8),
        b_gat.reshape(1, _D), g1.reshape(1, _D), beta1.reshape(1, _D),
        W1, b1.reshape(1, 2 * _D), W2, b2.reshape(1, _D),
        g2.reshape(1, _D), beta2.reshape(1, _D))
